# baseline (device time: 123604 ns/iter reference)
import jax
import jax.numpy as jnp
from jax import lax
from jax.experimental import pallas as pl
from jax.experimental.pallas import tpu as pltpu

BM = 512
CDT = jnp.bfloat16


def kernel(x):
    M, N = x.shape
    n_chunks = M // BM

    def body(x_hbm, out_hbm, vbuf, obuf, trow, brow, recv_row,
             colblk, send_col, recv_col,
             send_sems, recv_sems, sem_in, sem_out, sem_col):
        sx = lax.axis_index("x")
        sy = lax.axis_index("y")

        barrier = pltpu.get_barrier_semaphore()
        pl.semaphore_signal(barrier, inc=1, device_id=(1 - sx, sy),
                            device_id_type=pl.DeviceIdType.MESH)
        pl.semaphore_signal(barrier, inc=1, device_id=(sx, 1 - sy),
                            device_id_type=pl.DeviceIdType.MESH)
        pl.semaphore_wait(barrier, 2)

        row_idx = jnp.where(sx == 0, M - 1, 0)
        rdma_row = pltpu.make_async_remote_copy(
            src_ref=x_hbm.at[pl.ds(row_idx, 1), :],
            dst_ref=recv_row,
            send_sem=send_sems.at[0],
            recv_sem=recv_sems.at[0],
            device_id=(1 - sx, sy),
            device_id_type=pl.DeviceIdType.MESH,
        )
        rdma_row.start()

        @pl.when(sy == 0)
        def _():
            pltpu.make_async_copy(
                x_hbm.at[:, pl.ds(N - 128, 128)], colblk, sem_col).start()

        @pl.when(sy == 1)
        def _():
            pltpu.make_async_copy(
                x_hbm.at[:, pl.ds(0, 128)], colblk, sem_col).start()

        pltpu.make_async_copy(colblk, colblk, sem_col).wait()

        @pl.when(sy == 0)
        def _():
            send_col[...] = colblk[:, 127:128]

        @pl.when(sy == 1)
        def _():
            send_col[...] = colblk[:, 0:1]

        rdma_col = pltpu.make_async_remote_copy(
            src_ref=send_col,
            dst_ref=recv_col,
            send_sem=send_sems.at[1],
            recv_sem=recv_sems.at[1],
            device_id=(sx, 1 - sy),
            device_id_type=pl.DeviceIdType.MESH,
        )
        rdma_col.start()
        rdma_row.wait()
        rdma_col.wait()

        def start_in(c, slot):
            r0 = pl.multiple_of(c * BM, BM)
            pltpu.make_async_copy(
                x_hbm.at[pl.ds(r0, BM), :], vbuf.at[slot], sem_in.at[slot]
            ).start()

            @pl.when(c > 0)
            def _():
                pltpu.make_async_copy(
                    x_hbm.at[pl.ds(r0 - 1, 1), :], trow.at[slot],
                    sem_in.at[slot]).start()

            @pl.when(c == 0)
            def _():
                pltpu.make_async_copy(
                    recv_row, trow.at[slot], sem_in.at[slot]).start()

            @pl.when(c < n_chunks - 1)
            def _():
                pltpu.make_async_copy(
                    x_hbm.at[pl.ds(r0 + BM, 1), :], brow.at[slot],
                    sem_in.at[slot]).start()

            @pl.when(c == n_chunks - 1)
            def _():
                pltpu.make_async_copy(
                    recv_row, brow.at[slot], sem_in.at[slot]).start()

        start_in(0, 0)

        def chunk(c, _):
            slot = lax.rem(c, 2)

            @pl.when(c < n_chunks - 1)
            def _():
                start_in(c + 1, lax.rem(c + 1, 2))

            pltpu.make_async_copy(
                vbuf.at[slot], vbuf.at[slot], sem_in.at[slot]).wait()
            pltpu.make_async_copy(
                trow.at[slot], trow.at[slot], sem_in.at[slot]).wait()
            pltpu.make_async_copy(
                brow.at[slot], brow.at[slot], sem_in.at[slot]).wait()

            @pl.when(c >= 2)
            def _():
                pltpu.make_async_copy(
                    obuf.at[slot], obuf.at[slot], sem_out.at[slot]).wait()

            r0 = pl.multiple_of(c * BM, BM)
            vx = vbuf[slot]

            vb = vx.astype(CDT)
            nr = pltpu.roll(vb, 1, 0)
            sr = pltpu.roll(vb, BM - 1, 0)
            wr = pltpu.roll(vb, 1, 1)
            er = pltpu.roll(vb, N - 1, 1)
            half = jnp.asarray(0.5, CDT)
            eighth = jnp.asarray(0.125, CDT)
            obuf[slot] = (half * vb
                          + eighth * ((nr + sr) + (wr + er))).astype(jnp.float32)

            t = trow[slot]
            b = brow[slot]
            wr0 = jnp.concatenate(
                [recv_col[pl.ds(r0, 1), :], vx[0:1, :N - 1]], axis=1)
            er0 = jnp.concatenate(
                [vx[0:1, 1:], recv_col[pl.ds(r0, 1), :]], axis=1)
            obuf[slot, 0:1, :] = (0.5 * vx[0:1, :]
                                  + 0.125 * (t + vx[1:2, :] + wr0 + er0))
            wr1 = jnp.concatenate(
                [recv_col[pl.ds(r0 + BM - 1, 1), :], vx[BM - 1:BM, :N - 1]],
                axis=1)
            er1 = jnp.concatenate(
                [vx[BM - 1:BM, 1:], recv_col[pl.ds(r0 + BM - 1, 1), :]],
                axis=1)
            obuf[slot, BM - 1:BM, :] = (0.5 * vx[BM - 1:BM, :]
                                        + 0.125 * (vx[BM - 2:BM - 1, :] + b
                                                   + wr1 + er1))

            hcol = recv_col[pl.ds(r0, BM), :]
            n0 = jnp.concatenate([t[:, 0:1], vx[:BM - 1, 0:1]], axis=0)
            s0 = jnp.concatenate([vx[1:, 0:1], b[:, 0:1]], axis=0)
            obuf[slot, :, 0:1] = (0.5 * vx[:, 0:1]
                                  + 0.125 * (n0 + s0 + hcol + vx[:, 1:2]))
            n1 = jnp.concatenate([t[:, N - 1:N], vx[:BM - 1, N - 1:N]], axis=0)
            s1 = jnp.concatenate([vx[1:, N - 1:N], b[:, N - 1:N]], axis=0)
            obuf[slot, :, N - 1:N] = (0.5 * vx[:, N - 1:N]
                                      + 0.125 * (n1 + s1
                                                 + vx[:, N - 2:N - 1] + hcol))

            @pl.when(sy == 0)
            def _():
                obuf[slot, :, 0:1] = vx[:, 0:1]

            @pl.when(sy == 1)
            def _():
                obuf[slot, :, N - 1:N] = vx[:, N - 1:N]

            @pl.when((c == 0) & (sx == 0))
            def _():
                obuf[slot, 0:1, :] = vx[0:1, :]

            @pl.when((c == n_chunks - 1) & (sx == 1))
            def _():
                obuf[slot, BM - 1:BM, :] = vx[BM - 1:BM, :]

            pltpu.make_async_copy(
                obuf.at[slot], out_hbm.at[pl.ds(r0, BM), :], sem_out.at[slot]
            ).start()
            return 0

        lax.fori_loop(0, n_chunks, chunk, 0)

        pltpu.make_async_copy(obuf.at[0], obuf.at[0], sem_out.at[0]).wait()
        pltpu.make_async_copy(obuf.at[1], obuf.at[1], sem_out.at[1]).wait()

    return pl.pallas_call(
        body,
        out_shape=jax.ShapeDtypeStruct((M, N), jnp.float32),
        in_specs=[pl.BlockSpec(memory_space=pl.ANY)],
        out_specs=pl.BlockSpec(memory_space=pl.ANY),
        scratch_shapes=[
            pltpu.VMEM((2, BM, N), jnp.float32),
            pltpu.VMEM((2, BM, N), jnp.float32),
            pltpu.VMEM((2, 1, N), jnp.float32),
            pltpu.VMEM((2, 1, N), jnp.float32),
            pltpu.VMEM((1, N), jnp.float32),
            pltpu.VMEM((M, 128), jnp.float32),
            pltpu.VMEM((M, 1), jnp.float32),
            pltpu.VMEM((M, 1), jnp.float32),
            pltpu.SemaphoreType.DMA((2,)),
            pltpu.SemaphoreType.DMA((2,)),
            pltpu.SemaphoreType.DMA((2,)),
            pltpu.SemaphoreType.DMA((2,)),
            pltpu.SemaphoreType.DMA,
        ],
        compiler_params=pltpu.CompilerParams(
            collective_id=0, vmem_limit_bytes=100 * 1024 * 1024),
    )(x)
